# baseline (speedup 1.0000x reference)
import jax
import jax.numpy as jnp
from jax import lax
from jax.experimental import pallas as pl
from jax.experimental.pallas import tpu as pltpu

D = 128
HID = 10


def _rot(x, k):
    return jnp.concatenate([x[:, D - k:], x[:, :D - k]], axis=1)


def _loss_kernel(n2_ref, c2_ref, dp_ref, w_ref, b_ref, out_ref,
                 w1_s, w2_s, wd_s, c_s):
    j = pl.program_id(1)

    @pl.when(j == 0)
    def _prep():
        wg = w_ref[0]
        w1 = w_ref[1]
        w2 = w_ref[2]

        lane64 = lax.broadcasted_iota(jnp.int32, (64, D), 1)
        w1s = w1[1:65, :]
        wgn = wg[0:64, 0:1]
        wgc = wg[64:D, 0:1]
        top_n = jnp.where(lane64 == 10, wgn, 0.0)
        bot_n = jnp.where(lane64 == 11, wgn, 0.0)
        w1_s[0] = jnp.concatenate([top_n, bot_n], axis=0).astype(jnp.bfloat16)
        top_c = w1s + _rot(w1s, 16) + jnp.where(lane64 == 10, wgc, 0.0)
        bot_c = _rot(w1s, 32) + _rot(w1s, 48) + jnp.where(lane64 == 11, wgc,
                                                          0.0)
        w1_s[1] = jnp.concatenate([top_c, bot_c], axis=0).astype(jnp.bfloat16)

        w1r0 = w1[0:1, :]
        wd_s[...] = jnp.concatenate(
            [w1r0, _rot(w1r0, 32), jnp.zeros((6, D), jnp.float32)], axis=0)

        lane128 = lax.broadcasted_iota(jnp.int32, (D, D), 1)
        w2c = w2[:, 0:1]
        s16 = jnp.concatenate([w2c[D - 16:, :], w2c[:D - 16, :]], axis=0)
        s32 = jnp.concatenate([w2c[D - 32:, :], w2c[:D - 32, :]], axis=0)
        s48 = jnp.concatenate([w2c[D - 48:, :], w2c[:D - 48, :]], axis=0)
        w2_s[...] = (jnp.where(lane128 == 1, w2c, 0.0)
                     + jnp.where(lane128 == 0, s16, 0.0)
                     + jnp.where(lane128 == 2, s16, 0.0)
                     + jnp.where(lane128 == 4, s32, 0.0)
                     + jnp.where(lane128 == 3, s48, 0.0)
                     + jnp.where(lane128 == 5, s48, 0.0)).astype(jnp.bfloat16)

        lane1 = lax.broadcasted_iota(jnp.int32, (1, D), 1)
        b1r = b_ref[1:2, :]
        bg0 = b_ref[0:1, 0:1]
        b20 = b_ref[2:3, 0:1]
        bb = b1r + bg0 * w1r0
        c_s[...] = jnp.concatenate([
            _rot(w1r0, 16),
            _rot(w1r0, 48),
            b1r + _rot(b1r, 32) + _rot(bb, 16) + _rot(bb, 48),
            jnp.where(lane1 < 6, b20, 0.0),
            jnp.zeros((4, D), jnp.float32)], axis=0)

    m1 = jnp.dot(n2_ref[...], w1_s[0], preferred_element_type=jnp.float32)
    m1 = m1 + jnp.dot(c2_ref[...], w1_s[1], preferred_element_type=jnp.float32)
    m1 = m1 + jnp.dot(dp_ref[...], wd_s[0:2, :],
                      preferred_element_type=jnp.float32)

    gen_e = m1[:, 10:11]
    gen_o = m1[:, 11:12]
    pre = m1 + gen_e * c_s[0:1, :] + gen_o * c_s[1:2, :] + c_s[2:3, :]
    h = jnp.maximum(pre, 0.0)

    o = jnp.dot(h.astype(jnp.bfloat16), w2_s[...],
                preferred_element_type=jnp.float32) + c_s[3:4, :]

    log_d = jnp.minimum(o, 0.0) - jnp.log(1.0 + jnp.exp(-jnp.abs(o)))
    lane = lax.broadcasted_iota(jnp.int32, o.shape, 1)
    l1md = log_d - o
    v = jnp.where(lane == 2, l1md, jnp.where(lane == 5, l1md, log_d))
    part = jnp.broadcast_to(jnp.sum(v, axis=0, keepdims=True), (8, D))

    @pl.when(j == 0)
    def _():
        out_ref[...] = part

    @pl.when(j != 0)
    def _():
        out_ref[...] += part


def kernel(noise, data, cond, w_packed, b_packed):
    B = noise.shape[0]

    n2 = jnp.reshape(noise, (B // 2, D)).astype(jnp.bfloat16)
    c2 = jnp.reshape(cond, (B // 2, D)).astype(jnp.bfloat16)
    dp = jnp.reshape(data, (B // 2, 2))

    half = B // 4
    R = min(2048, half)
    S = half // R

    out = pl.pallas_call(
        _loss_kernel,
        out_shape=jax.ShapeDtypeStruct((16, D), jnp.float32),
        grid=(2, S),
        in_specs=[
            pl.BlockSpec((R, D), lambda i, j: (i * S + j, 0)),
            pl.BlockSpec((R, D), lambda i, j: (i * S + j, 0)),
            pl.BlockSpec((R, 2), lambda i, j: (i * S + j, 0)),
            pl.BlockSpec((3, D, D), lambda i, j: (0, 0, 0)),
            pl.BlockSpec((8, D), lambda i, j: (0, 0)),
        ],
        out_specs=pl.BlockSpec((8, D), lambda i, j: (i, 0)),
        scratch_shapes=[
            pltpu.VMEM((2, D, D), jnp.bfloat16),
            pltpu.VMEM((D, D), jnp.bfloat16),
            pltpu.VMEM((8, D), jnp.float32),
            pltpu.VMEM((8, D), jnp.float32),
        ],
        compiler_params=pltpu.CompilerParams(
            dimension_semantics=("parallel", "arbitrary")),
    )(n2, c2, dp, w_packed, b_packed)

    acc = out[0, :] + out[8, :]
    inv_b = 1.0 / B
    gen_loss = -(acc[0] + acc[3]) * inv_b
    disc_loss = -(acc[1] + acc[4] + acc[2] + acc[5]) * inv_b
    return gen_loss, disc_loss


# speedup vs baseline: 1.4398x; 1.4398x over previous
import jax
import jax.numpy as jnp
from jax import lax
from jax.experimental import pallas as pl
from jax.experimental.pallas import tpu as pltpu

D = 128
HID = 10


def _loss_kernel(noise_ref, cond_ref, data_ref, w_ref, b_ref, out_ref,
                 wf_s, w2p_s, c_s):
    j = pl.program_id(1)

    @pl.when(j == 0)
    def _prep():
        wg = w_ref[0]
        w1 = w_ref[1]
        w2 = w_ref[2]

        lane64 = lax.broadcasted_iota(jnp.int32, (64, D), 1)
        w1s = w1[1:65, :]
        w1s16 = jnp.concatenate([w1s[:, D - 16:], w1s[:, :D - 16]], axis=1)
        noise_part = jnp.where(lane64 == HID, wg[0:64, 0:1], 0.0)
        cond_part = w1s + w1s16 + jnp.where(lane64 == HID, wg[64:D, 0:1], 0.0)
        wf_s[...] = jnp.concatenate([noise_part, cond_part],
                                    axis=0).astype(jnp.bfloat16)

        lane128 = lax.broadcasted_iota(jnp.int32, (D, D), 1)
        w2c = w2[:, 0:1]
        w2sh = jnp.concatenate([w2c[D - 16:, :], w2c[:D - 16, :]], axis=0)
        w2p_s[...] = (jnp.where(lane128 == 1, w2c, 0.0)
                      + jnp.where(lane128 == 0, w2sh, 0.0)
                      + jnp.where(lane128 == 2, w2sh, 0.0)).astype(jnp.bfloat16)

        lane1 = lax.broadcasted_iota(jnp.int32, (1, D), 1)
        w1r0 = w1[0:1, :]
        w1r0h = jnp.concatenate([w1r0[:, D - 16:], w1r0[:, :D - 16]], axis=1)
        b1r = b_ref[1:2, :]
        b1rh = jnp.concatenate([b1r[:, D - 16:], b1r[:, :D - 16]], axis=1)
        bg0 = b_ref[0:1, 0:1]
        b20 = b_ref[2:3, 0:1]
        c_s[...] = jnp.concatenate([
            w1r0,
            w1r0h,
            b1r + b1rh + bg0 * w1r0h,
            jnp.where(lane1 < 3, b20, 0.0),
            jnp.zeros((4, D), jnp.float32)], axis=0)

    m1 = jnp.dot(noise_ref[...], wf_s[0:64, :],
                 preferred_element_type=jnp.float32)
    m1 = m1 + jnp.dot(cond_ref[...], wf_s[64:D, :],
                      preferred_element_type=jnp.float32)

    genp = m1[:, HID:HID + 1]
    pre = (m1 + genp * c_s[1:2, :]
           + data_ref[...] * c_s[0:1, :]
           + c_s[2:3, :])
    h = jnp.maximum(pre, 0.0)

    o = jnp.dot(h.astype(jnp.bfloat16), w2p_s[...],
                preferred_element_type=jnp.float32) + c_s[3:4, :]

    log_d = jnp.minimum(o, 0.0) - jnp.log(1.0 + jnp.exp(-jnp.abs(o)))
    lane = lax.broadcasted_iota(jnp.int32, o.shape, 1)
    v = jnp.where(lane == 2, log_d - o, log_d)
    part = jnp.broadcast_to(jnp.sum(v, axis=0, keepdims=True), (8, D))

    @pl.when(j == 0)
    def _():
        out_ref[...] = part

    @pl.when(j != 0)
    def _():
        out_ref[...] += part


def kernel(noise, data, cond, w_packed, b_packed):
    B = noise.shape[0]
    nd = noise.shape[1]
    cd = cond.shape[1]

    noise16 = noise.astype(jnp.bfloat16)
    cond16 = cond.astype(jnp.bfloat16)

    half = B // 2
    R = min(8192, half)
    S = half // R

    out = pl.pallas_call(
        _loss_kernel,
        out_shape=jax.ShapeDtypeStruct((16, D), jnp.float32),
        grid=(2, S),
        in_specs=[
            pl.BlockSpec((R, nd), lambda i, j: (i * S + j, 0)),
            pl.BlockSpec((R, cd), lambda i, j: (i * S + j, 0)),
            pl.BlockSpec((R, 1), lambda i, j: (i * S + j, 0)),
            pl.BlockSpec((3, D, D), lambda i, j: (0, 0, 0)),
            pl.BlockSpec((8, D), lambda i, j: (0, 0)),
        ],
        out_specs=pl.BlockSpec((8, D), lambda i, j: (i, 0)),
        scratch_shapes=[
            pltpu.VMEM((D, D), jnp.bfloat16),
            pltpu.VMEM((D, D), jnp.bfloat16),
            pltpu.VMEM((8, D), jnp.float32),
        ],
        compiler_params=pltpu.CompilerParams(
            dimension_semantics=("parallel", "arbitrary")),
    )(noise16, cond16, data, w_packed, b_packed)

    acc = out[0, :] + out[8, :]
    inv_b = 1.0 / B
    gen_loss = -acc[0] * inv_b
    disc_loss = -(acc[1] + acc[2]) * inv_b
    return gen_loss, disc_loss


# speedup vs baseline: 1.4604x; 1.0143x over previous
import jax
import jax.numpy as jnp
from jax import lax
from jax.experimental import pallas as pl
from jax.experimental.pallas import tpu as pltpu

D = 128
HID = 10


def _loss_kernel(noise_ref, cond_ref, data_ref, w_ref, b_ref, out_ref,
                 wf_s, w2p_s, c_s):
    j = pl.program_id(1)

    @pl.when(j == 0)
    def _prep():
        wg = w_ref[0]
        w1 = w_ref[1]
        w2 = w_ref[2]

        lane64 = lax.broadcasted_iota(jnp.int32, (64, D), 1)
        w1s = w1[1:65, :]
        w1s16 = jnp.concatenate([w1s[:, D - 16:], w1s[:, :D - 16]], axis=1)
        noise_part = jnp.where(lane64 == HID, wg[0:64, 0:1], 0.0)
        cond_part = w1s + w1s16 + jnp.where(lane64 == HID, wg[64:D, 0:1], 0.0)
        wf_s[...] = jnp.concatenate([noise_part, cond_part],
                                    axis=0).astype(jnp.bfloat16)

        lane128 = lax.broadcasted_iota(jnp.int32, (D, D), 1)
        w2c = w2[:, 0:1]
        w2sh = jnp.concatenate([w2c[D - 16:, :], w2c[:D - 16, :]], axis=0)
        w2p_s[...] = (jnp.where(lane128 == 1, w2c, 0.0)
                      + jnp.where(lane128 == 0, w2sh, 0.0)
                      + jnp.where(lane128 == 2, w2sh, 0.0)).astype(jnp.bfloat16)

        lane1 = lax.broadcasted_iota(jnp.int32, (1, D), 1)
        w1r0 = w1[0:1, :]
        w1r0h = jnp.concatenate([w1r0[:, D - 16:], w1r0[:, :D - 16]], axis=1)
        b1r = b_ref[1:2, :]
        b1rh = jnp.concatenate([b1r[:, D - 16:], b1r[:, :D - 16]], axis=1)
        bg0 = b_ref[0:1, 0:1]
        b20 = b_ref[2:3, 0:1]
        c_s[...] = jnp.concatenate([
            w1r0,
            w1r0h,
            b1r + b1rh + bg0 * w1r0h,
            jnp.where(lane1 < 3, b20, 0.0),
            jnp.zeros((4, D), jnp.float32)], axis=0)

    m1 = jnp.dot(noise_ref[...], wf_s[0:64, :],
                 preferred_element_type=jnp.float32)
    m1 = m1 + jnp.dot(cond_ref[...], wf_s[64:D, :],
                      preferred_element_type=jnp.float32)

    genp = m1[:, HID:HID + 1]
    pre = (m1 + genp * c_s[1:2, :]
           + data_ref[...] * c_s[0:1, :]
           + c_s[2:3, :])
    h = jnp.maximum(pre, 0.0)

    o = jnp.dot(h.astype(jnp.bfloat16), w2p_s[...],
                preferred_element_type=jnp.float32) + c_s[3:4, :]

    log_d = jnp.minimum(o, 0.0) - jnp.log(1.0 + jnp.exp(-jnp.abs(o)))
    lane = lax.broadcasted_iota(jnp.int32, o.shape, 1)
    v = jnp.where(lane == 2, log_d - o, log_d)
    part = jnp.broadcast_to(jnp.sum(v, axis=0, keepdims=True), (8, D))

    @pl.when(j == 0)
    def _():
        out_ref[...] = part

    @pl.when(j != 0)
    def _():
        out_ref[...] += part


def kernel(noise, data, cond, w_packed, b_packed):
    B = noise.shape[0]
    nd = noise.shape[1]
    cd = cond.shape[1]

    noise16 = noise.astype(jnp.bfloat16)
    cond16 = cond.astype(jnp.bfloat16)

    half = B // 2
    R = min(4096, half)
    S = half // R

    out = pl.pallas_call(
        _loss_kernel,
        out_shape=jax.ShapeDtypeStruct((16, D), jnp.float32),
        grid=(2, S),
        in_specs=[
            pl.BlockSpec((R, nd), lambda i, j: (i * S + j, 0)),
            pl.BlockSpec((R, cd), lambda i, j: (i * S + j, 0)),
            pl.BlockSpec((R, 1), lambda i, j: (i * S + j, 0)),
            pl.BlockSpec((3, D, D), lambda i, j: (0, 0, 0)),
            pl.BlockSpec((8, D), lambda i, j: (0, 0)),
        ],
        out_specs=pl.BlockSpec((8, D), lambda i, j: (i, 0)),
        scratch_shapes=[
            pltpu.VMEM((D, D), jnp.bfloat16),
            pltpu.VMEM((D, D), jnp.bfloat16),
            pltpu.VMEM((8, D), jnp.float32),
        ],
        compiler_params=pltpu.CompilerParams(
            dimension_semantics=("parallel", "arbitrary")),
    )(noise16, cond16, data, w_packed, b_packed)

    acc = out[0, :] + out[8, :]
    inv_b = 1.0 / B
    gen_loss = -acc[0] * inv_b
    disc_loss = -(acc[1] + acc[2]) * inv_b
    return gen_loss, disc_loss
